# baseline (speedup 1.0000x reference)
import functools

import jax
import jax.numpy as jnp
from jax import lax
from jax.experimental import pallas as pl
from jax.experimental.pallas import tpu as pltpu
from jax.experimental.pallas import tpu_sc as plsc

_VOCAB = 1_000_000
_D = 64
_B = 4096 * 200
_NW = 32
_BPW = _B // _NW
_CH = 512
_NCH = _BPW // _CH

_mesh = plsc.VectorSubcoreMesh(core_axis_name="c", subcore_axis_name="s")


@functools.partial(
    pl.kernel,
    out_type=jax.ShapeDtypeStruct((_B, _D), jnp.float32),
    mesh=_mesh,
    scratch_types=[
        pltpu.VMEM((_CH,), jnp.int32),
        pltpu.VMEM((_CH, _D), jnp.float32),
        pltpu.SemaphoreType.DMA,
    ],
    compiler_params=pltpu.CompilerParams(use_tc_tiling_on_sc=False),
)
def _gather_kernel(idx_hbm, table_hbm, out_hbm, idx_v, rows_v, sem):
    wid = lax.axis_index("s") * 2 + lax.axis_index("c")
    base = wid * _BPW

    def body(i, carry):
        off = base + i * _CH
        pltpu.sync_copy(idx_hbm.at[pl.ds(off, _CH)], idx_v)
        pltpu.async_copy(table_hbm.at[idx_v], rows_v, sem).wait()
        pltpu.sync_copy(rows_v, out_hbm.at[pl.ds(off, _CH)])
        return carry

    lax.fori_loop(0, _NCH, body, 0)


def kernel(x, embedding_table, training, mask):
    idx = x.reshape(-1).astype(jnp.int32)
    out = _gather_kernel(idx, embedding_table)
    return out.reshape(x.shape[0], x.shape[1], _D)


# speedup vs baseline: 1.0461x; 1.0461x over previous
import functools

import jax
import jax.numpy as jnp
from jax import lax
from jax.experimental import pallas as pl
from jax.experimental.pallas import tpu as pltpu
from jax.experimental.pallas import tpu_sc as plsc

_VOCAB = 1_000_000
_D = 64
_B = 4096 * 200
_NW = 32
_BPW = _B // _NW
_CH = 512
_NCH = _BPW // _CH

_mesh = plsc.VectorSubcoreMesh(core_axis_name="c", subcore_axis_name="s")


_NBUF = 2
_NSTEP = _NCH // _NBUF


@functools.partial(
    pl.kernel,
    out_type=jax.ShapeDtypeStruct((_B, _D), jnp.float32),
    mesh=_mesh,
    scratch_types=[
        [pltpu.VMEM((_CH,), jnp.int32) for _ in range(_NBUF)],
        [pltpu.VMEM((_CH, _D), jnp.float32) for _ in range(_NBUF)],
        [pltpu.SemaphoreType.DMA for _ in range(_NBUF)],
        [pltpu.SemaphoreType.DMA for _ in range(_NBUF)],
        [pltpu.SemaphoreType.DMA for _ in range(_NBUF)],
    ],
    compiler_params=pltpu.CompilerParams(use_tc_tiling_on_sc=False),
)
def _gather_kernel(idx_hbm, table_hbm, out_hbm, idx_v, rows_v, sem_i, sem_g, sem_s):
    wid = lax.axis_index("s") * 2 + lax.axis_index("c")
    base = wid * _BPW

    def idx_copy(b, off):
        return pltpu.make_async_copy(
            idx_hbm.at[pl.ds(off, _CH)], idx_v[b], sem_i[b])

    def gather_copy(b):
        return pltpu.make_async_copy(table_hbm.at[idx_v[b]], rows_v[b], sem_g[b])

    def store_copy(b, off):
        return pltpu.make_async_copy(
            rows_v[b], out_hbm.at[pl.ds(off, _CH)], sem_s[b])

    for b in range(_NBUF):
        idx_copy(b, base + b * _CH).start()
    for b in range(_NBUF):
        idx_copy(b, base + b * _CH).wait()
        gather_copy(b).start()

    def body(g, carry):
        for b in range(_NBUF):
            off_prev = base + ((g - 1) * _NBUF + b) * _CH
            off_new = base + (g * _NBUF + b) * _CH
            gather_copy(b).wait()
            store_copy(b, off_prev).start()
            idx_copy(b, off_new).start()
            store_copy(b, off_prev).wait()
            idx_copy(b, off_new).wait()
            gather_copy(b).start()
        return carry

    lax.fori_loop(1, _NSTEP, body, 0)

    for b in range(_NBUF):
        off = base + ((_NSTEP - 1) * _NBUF + b) * _CH
        gather_copy(b).wait()
        store_copy(b, off).start()
    for b in range(_NBUF):
        off = base + ((_NSTEP - 1) * _NBUF + b) * _CH
        store_copy(b, off).wait()


def kernel(x, embedding_table, training, mask):
    idx = x.reshape(-1).astype(jnp.int32)
    out = _gather_kernel(idx, embedding_table)
    return out.reshape(x.shape[0], x.shape[1], _D)
